# baseline (speedup 1.0000x reference)
import jax
import jax.numpy as jnp
from jax import lax
from jax.experimental import pallas as pl


def _sigmoid(x):
    return 1.0 / (1.0 + jnp.exp(-x))


def _leaky(x):
    return jnp.where(x >= 0, x, 0.01 * x)


def _fused_body(data_ref, d_ref, ei_ref,
                Ws0_ref, Wm0_ref, Wg0_ref, bg0_ref, b0_ref,
                Ws1_ref, Wm1_ref, Wg1_ref, bg1_ref, b1_ref,
                Ws2_ref, Wm2_ref, Wg2_ref, bg2_ref, b2_ref,
                Wf01_ref, bf0_ref, bf1_ref,
                Wf2_ref, bf2_ref, Wf3_ref, bf3_ref,
                out_ref):
    Wf0_ref = Wf01_ref.at[0:220, :]
    Wf1_ref = Wf01_ref.at[220:348, :]
    E = 90
    N = 10
    ei = ei_ref[...]
    node_iota = lax.broadcasted_iota(jnp.int32, (N, E), 0)
    ST = (ei[0:1, :] == node_iota).astype(jnp.float32)
    DT = (ei[1:1 + 1, :] == node_iota).astype(jnp.float32)

    def layer(x, Ws, Wm, Wg, bg, b, din, dout):
        Wcat = jnp.concatenate([Wg[:din, :], Wg[din:, :], Wm], axis=1)
        P = jnp.dot(x, Wcat, preferred_element_type=jnp.float32)
        Pd = lax.dot_general(DT, P[:, 0:1],
                             (((0,), (0,)), ((), ())),
                             preferred_element_type=jnp.float32)
        Ps = lax.dot_general(ST, P[:, 1:],
                             (((0,), (0,)), ((), ())),
                             preferred_element_type=jnp.float32)
        gate = _sigmoid(Pd + Ps[:, 0:1] + bg[0])
        msg = gate * Ps[:, 1:]
        agg = lax.dot_general(DT, msg,
                              (((1,), (0,)), ((), ())),
                              preferred_element_type=jnp.float32)
        xself = jnp.dot(x, Ws, preferred_element_type=jnp.float32)
        h = jnp.concatenate([xself, agg], axis=1) + b[None, :]
        return _leaky(h)

    x = layer(data_ref[...], Ws0_ref[...], Wm0_ref[...], Wg0_ref[...],
              bg0_ref[...], b0_ref[...], 1, 5)
    x = layer(x, Ws1_ref[...], Wm1_ref[...], Wg1_ref[...],
              bg1_ref[...], b1_ref[...], 10, 5)
    x = layer(x, Ws2_ref[...], Wm2_ref[...], Wg2_ref[...],
              bg2_ref[...], b2_ref[...], 10, 5)

    def row_flatten(a, cols):
        rep = jnp.concatenate([a] * N, axis=1)
        k_iota = lax.broadcasted_iota(jnp.int32, (N, N * cols), 1)
        n_iota = lax.broadcasted_iota(jnp.int32, (N, N * cols), 0)
        mask = (k_iota // cols) == n_iota
        spread = jnp.where(mask, rep, 0.0)
        ones = jnp.ones((1, N), jnp.float32)
        return jnp.dot(ones, spread, preferred_element_type=jnp.float32)

    x_flat = row_flatten(x, 10)
    d_flat = row_flatten(d_ref[...], 12)
    flat = jnp.concatenate([x_flat, d_flat], axis=1)

    h = _leaky(jnp.dot(flat, Wf0_ref[...], preferred_element_type=jnp.float32)
               + bf0_ref[...][None, :])
    h = _leaky(jnp.dot(h, Wf1_ref[...], preferred_element_type=jnp.float32)
               + bf1_ref[...][None, :])
    h = _leaky(jnp.dot(h, Wf2_ref[...], preferred_element_type=jnp.float32)
               + bf2_ref[...][None, :])
    h = _sigmoid(jnp.dot(h, Wf3_ref[...], preferred_element_type=jnp.float32)
                 + bf3_ref[...][None, :])
    out_ref[...] = h


def kernel(data, d, edge_index, Ws0, Wm0, Wg0, bg0, b0, Ws1, Wm1, Wg1, bg1, b1,
           Ws2, Wm2, Wg2, bg2, b2, Wf0, bf0, Wf1, bf1, Wf2, bf2, Wf3, bf3):
    out = pl.pallas_call(
        _fused_body,
        out_shape=jax.ShapeDtypeStruct((1, 2), jnp.float32),
    )(data, d, edge_index.astype(jnp.int32), Ws0, Wm0, Wg0, bg0, b0,
      Ws1, Wm1, Wg1, bg1, b1, Ws2, Wm2, Wg2, bg2, b2,
      jnp.concatenate([Wf0, Wf1], axis=0), bf0, bf1, Wf2, bf2, Wf3, bf3)
    return out.reshape(2)


# speedup vs baseline: 1.4882x; 1.4882x over previous
import jax
import jax.numpy as jnp
from jax import lax
from jax.experimental import pallas as pl


def _sigmoid(x):
    return 1.0 / (1.0 + jnp.exp(-x))


def _leaky(x):
    return jnp.where(x >= 0, x, 0.01 * x)


def _fused_body(data_ref, d_ref, ei_ref,
                Ws0_ref, Wm0_ref, Wg0_ref, bg0_ref, b0_ref,
                Ws1_ref, Wm1_ref, Wg1_ref, bg1_ref, b1_ref,
                Ws2_ref, Wm2_ref, Wg2_ref, bg2_ref, b2_ref,
                Wf0_ref, bf0_ref, Wf1_ref, bf1_ref,
                Wf2_ref, bf2_ref, Wf3_ref, bf3_ref,
                out_ref):
    E = 90
    N = 10
    ei = ei_ref[...]
    node_iota = lax.broadcasted_iota(jnp.int32, (N, E), 0)
    ST = (ei[0:1, :] == node_iota).astype(jnp.float32)
    DT = (ei[1:1 + 1, :] == node_iota).astype(jnp.float32)

    def layer(x, Ws, Wm, Wg, bg, b, din, dout):
        Wcat = jnp.concatenate([Wg[:din, :], Wg[din:, :], Wm], axis=1)
        P = jnp.dot(x, Wcat, preferred_element_type=jnp.float32)
        Pd = lax.dot_general(DT, P[:, 0:1],
                             (((0,), (0,)), ((), ())),
                             preferred_element_type=jnp.float32)
        Ps = lax.dot_general(ST, P[:, 1:],
                             (((0,), (0,)), ((), ())),
                             preferred_element_type=jnp.float32)
        gate = _sigmoid(Pd + Ps[:, 0:1] + bg[0, 0])
        msg = gate * Ps[:, 1:]
        agg = lax.dot_general(DT, msg,
                              (((1,), (0,)), ((), ())),
                              preferred_element_type=jnp.float32)
        xself = jnp.dot(x, Ws, preferred_element_type=jnp.float32)
        h = jnp.concatenate([xself, agg], axis=1) + b
        return _leaky(h)

    x = layer(data_ref[...], Ws0_ref[...], Wm0_ref[...], Wg0_ref[...],
              bg0_ref[...], b0_ref[...], 1, 5)
    x = layer(x, Ws1_ref[...], Wm1_ref[...], Wg1_ref[...],
              bg1_ref[...], b1_ref[...], 10, 5)
    x = layer(x, Ws2_ref[...], Wm2_ref[...], Wg2_ref[...],
              bg2_ref[...], b2_ref[...], 10, 5)

    def row_flatten(a, cols):
        rep = jnp.concatenate([a] * N, axis=1)
        k_iota = lax.broadcasted_iota(jnp.int32, (N, N * cols), 1)
        n_iota = lax.broadcasted_iota(jnp.int32, (N, N * cols), 0)
        mask = (k_iota // cols) == n_iota
        spread = jnp.where(mask, rep, 0.0)
        ones = jnp.ones((1, N), jnp.float32)
        return jnp.dot(ones, spread, preferred_element_type=jnp.float32)

    x_flat = row_flatten(x, 10)
    d_flat = row_flatten(d_ref[...], 12)
    flat = jnp.concatenate([x_flat, d_flat], axis=1)

    h = _leaky(jnp.dot(flat, Wf0_ref[...], preferred_element_type=jnp.float32)
               + bf0_ref[...])
    h = _leaky(jnp.dot(h, Wf1_ref[...], preferred_element_type=jnp.float32)
               + bf1_ref[...])
    h = _leaky(jnp.dot(h, Wf2_ref[...], preferred_element_type=jnp.float32)
               + bf2_ref[...])
    h = _sigmoid(jnp.dot(h, Wf3_ref[...], preferred_element_type=jnp.float32)
                 + bf3_ref[...])
    out_ref[...] = h


def kernel(data, d, edge_index, Ws0, Wm0, Wg0, bg0, b0, Ws1, Wm1, Wg1, bg1, b1,
           Ws2, Wm2, Wg2, bg2, b2, Wf0, bf0, Wf1, bf1, Wf2, bf2, Wf3, bf3):
    out = pl.pallas_call(
        _fused_body,
        out_shape=jax.ShapeDtypeStruct((1, 2), jnp.float32),
    )(data, d, edge_index.astype(jnp.int32),
      Ws0, Wm0, Wg0, bg0.reshape(1, -1), b0.reshape(1, -1),
      Ws1, Wm1, Wg1, bg1.reshape(1, -1), b1.reshape(1, -1),
      Ws2, Wm2, Wg2, bg2.reshape(1, -1), b2.reshape(1, -1),
      Wf0, bf0.reshape(1, -1), Wf1, bf1.reshape(1, -1),
      Wf2, bf2.reshape(1, -1), Wf3, bf3.reshape(1, -1))
    return out.reshape(2)


# speedup vs baseline: 3.3759x; 2.2684x over previous
import jax
import jax.numpy as jnp
from jax import lax
from jax.experimental import pallas as pl


def _sigmoid(x):
    return 1.0 / (1.0 + jnp.exp(-x))


def _leaky(x):
    return jnp.where(x >= 0, x, 0.01 * x)


def _fused_body(dataT_ref, d_ref, ei_ref,
                Ws0_ref, Wm0_ref, Wg0T_ref, bg0_ref, b0_ref,
                Ws1T_ref, Wm1T_ref, Wg1T_ref, bg1_ref, b1_ref,
                Ws2T_ref, Wm2T_ref, Wg2T_ref, bg2_ref, b2_ref,
                Wf0_ref, bf0_ref, Wf1_ref, bf1_ref,
                Wf2T_ref, bf2_ref, Wf3T_ref, bf3_ref,
                out_ref):
    E = 90
    N = 10
    ei = ei_ref[...]
    node_iota = lax.broadcasted_iota(jnp.int32, (N, E), 0)
    ST = (ei[0:1, :] == node_iota).astype(jnp.float32)
    DT = (ei[1:2, :] == node_iota).astype(jnp.float32)

    def edges_and_agg(P, bg, b, dout):
        Pd = lax.dot_general(DT, P[:, 0:1],
                             (((0,), (0,)), ((), ())),
                             preferred_element_type=jnp.float32)
        Ps = lax.dot_general(ST, P[:, 1:2 + dout],
                             (((0,), (0,)), ((), ())),
                             preferred_element_type=jnp.float32)
        gate = _sigmoid(Pd + Ps[:, 0:1] + bg[0, 0])
        msg = gate * Ps[:, 1:]
        agg = lax.dot_general(DT, msg,
                              (((1,), (0,)), ((), ())),
                              preferred_element_type=jnp.float32)
        h = jnp.concatenate([P[:, 2 + dout:2 + 2 * dout], agg], axis=1) + b
        return _leaky(h)

    Wg0T = Wg0T_ref[...]
    Wcat0 = jnp.concatenate([Wg0T[:, 0:1], Wg0T[:, 1:2],
                             Wm0_ref[...], Ws0_ref[...]], axis=1)
    P = lax.dot_general(dataT_ref[...], Wcat0,
                        (((0,), (0,)), ((), ())),
                        preferred_element_type=jnp.float32)
    x = edges_and_agg(P, bg0_ref, b0_ref[...], 5)

    for WsT_ref, WmT_ref, WgT_ref, bg_ref, b_ref in (
            (Ws1T_ref, Wm1T_ref, Wg1T_ref, bg1_ref, b1_ref),
            (Ws2T_ref, Wm2T_ref, Wg2T_ref, bg2_ref, b2_ref)):
        WgT = WgT_ref[...]
        WcatT = jnp.concatenate([WgT[:, 0:N], WgT[:, N:2 * N],
                                 WmT_ref[...], WsT_ref[...]], axis=0)
        P = lax.dot_general(x, WcatT,
                            (((1,), (1,)), ((), ())),
                            preferred_element_type=jnp.float32)
        x = edges_and_agg(P, bg_ref, b_ref[...], 5)

    def row_flatten(a, cols):
        rep = jnp.concatenate([a] * N, axis=1)
        k_iota = lax.broadcasted_iota(jnp.int32, (N, N * cols), 1)
        n_iota = lax.broadcasted_iota(jnp.int32, (N, N * cols), 0)
        mask = (k_iota // cols) == n_iota
        spread = jnp.where(mask, rep, 0.0)
        ones = jnp.ones((1, N), jnp.float32)
        return jnp.dot(ones, spread, preferred_element_type=jnp.float32)

    x_flat = row_flatten(x, 10)
    d_flat = row_flatten(d_ref[...], 12)
    flat = jnp.concatenate([x_flat, d_flat], axis=1)

    h = _leaky(jnp.dot(flat, Wf0_ref[...], preferred_element_type=jnp.float32)
               + bf0_ref[...])
    h = _leaky(jnp.dot(h, Wf1_ref[...], preferred_element_type=jnp.float32)
               + bf1_ref[...])
    h = _leaky(lax.dot_general(h, Wf2T_ref[...],
                               (((1,), (1,)), ((), ())),
                               preferred_element_type=jnp.float32)
               + bf2_ref[...])
    h = _sigmoid(lax.dot_general(h, Wf3T_ref[...],
                                 (((1,), (1,)), ((), ())),
                                 preferred_element_type=jnp.float32)
                 + bf3_ref[...])
    out_ref[...] = h


def kernel(data, d, edge_index, Ws0, Wm0, Wg0, bg0, b0, Ws1, Wm1, Wg1, bg1, b1,
           Ws2, Wm2, Wg2, bg2, b2, Wf0, bf0, Wf1, bf1, Wf2, bf2, Wf3, bf3):
    out = pl.pallas_call(
        _fused_body,
        out_shape=jax.ShapeDtypeStruct((1, 2), jnp.float32),
    )(data.T, d, edge_index.astype(jnp.int32),
      Ws0, Wm0, Wg0.T, bg0.reshape(1, -1), b0.reshape(1, -1),
      Ws1.T, Wm1.T, Wg1.T, bg1.reshape(1, -1), b1.reshape(1, -1),
      Ws2.T, Wm2.T, Wg2.T, bg2.reshape(1, -1), b2.reshape(1, -1),
      Wf0, bf0.reshape(1, -1), Wf1, bf1.reshape(1, -1),
      Wf2.T, bf2.reshape(1, -1), Wf3.T, bf3.reshape(1, -1))
    return out.reshape(2)
